# baseline (speedup 1.0000x reference)
import jax
import jax.numpy as jnp
from jax import lax
from jax.experimental import pallas as pl
from jax.experimental.pallas import tpu as pltpu
from jax.experimental.pallas import tpu_sc as plsc

N_GRAPH, N_NODE, N_FEAT = 256, 256, 8
EMB = 64
NC, NS = 2, 16
NW = NC * NS
GPW = N_GRAPH // NW
ROWS = N_NODE + 1
E = N_NODE * N_FEAT
CHUNK = 256
NODES_PER_CHUNK = CHUNK // N_FEAT
N_CHUNK = E // CHUNK


def _sc_call(x_idx, deg_idx, node_table, degree_table, graph_token):
    mesh = plsc.VectorSubcoreMesh(
        core_axis_name="c", subcore_axis_name="s",
        num_cores=NC, num_subcores=NS)

    @pl.kernel(
        out_type=jax.ShapeDtypeStruct((N_GRAPH, ROWS, EMB), jnp.float32),
        mesh=mesh,
        scratch_types=[
            pltpu.VMEM((2, N_NODE, N_FEAT), jnp.int32),
            pltpu.VMEM((2, E), jnp.int32),
            pltpu.VMEM((2, N_NODE), jnp.int32),
            pltpu.VMEM((2, CHUNK, EMB), jnp.float32),
            pltpu.VMEM((N_NODE, EMB), jnp.float32),
            pltpu.VMEM((EMB,), jnp.float32),
            pltpu.VMEM((2, ROWS, EMB), jnp.float32),
            pltpu.SemaphoreType.DMA,
            pltpu.SemaphoreType.DMA,
            pltpu.SemaphoreType.DMA,
            pltpu.SemaphoreType.DMA,
            pltpu.SemaphoreType.DMA,
            pltpu.SemaphoreType.DMA,
            pltpu.SemaphoreType.DMA,
        ],
        compiler_params=pltpu.CompilerParams(use_tc_tiling_on_sc=False,
                                             needs_layout_passes=False),
    )
    def k(x_hbm, deg_hbm, tab_hbm, dtab_hbm, tok_hbm, out_hbm,
          idx_v, idxf_v, didx_v, rows_v, drows_v, tok_v, out_v,
          gsem0, gsem1, osem0, osem1, ixsem0, ixsem1, dgsem):
        gsem = (gsem0, gsem1)
        osem = (osem0, osem1)
        ixsem = (ixsem0, ixsem1)

        c = lax.axis_index("c")
        s = lax.axis_index("s")
        wid = c * NS + s

        pltpu.sync_copy(tok_hbm.at[0], tok_v)

        def fetch_idx(i, p):
            g = wid * GPW + i
            a = pltpu.async_copy(x_hbm.at[g], idx_v.at[p], ixsem[p])
            b = pltpu.async_copy(deg_hbm.at[g], didx_v.at[p], ixsem[p])
            return (a, b)

        idesc = [None, None]
        gdesc = [None, None]
        odesc = [None, None]

        idesc[0] = fetch_idx(0, 0)

        for i in range(GPW):
            p = i % 2
            g = wid * GPW + i

            idesc[p][0].wait()
            idesc[p][1].wait()
            ddesc = pltpu.async_copy(dtab_hbm.at[didx_v.at[p]],
                                     drows_v, dgsem)
            if i + 1 < GPW:
                idesc[1 - p] = fetch_idx(i + 1, 1 - p)

            lane = lax.iota(jnp.int32, 16)

            @plsc.parallel_loop(0, E // 16, unroll=4)
            def repack(j, p=p):
                e = j * 16 + lane
                v = plsc.load_gather(idx_v.at[p],
                                     [e >> 3, e & (N_FEAT - 1)])
                idxf_v[p, pl.ds(j * 16, 16)] = v

            gdesc[0] = pltpu.async_copy(
                tab_hbm.at[idxf_v.at[p].at[pl.ds(0, CHUNK)]],
                rows_v.at[0], gsem[0])
            if i >= 2:
                odesc[p].wait()
            for q in range(EMB // 16):
                out_v[p, 0, pl.ds(q * 16, 16)] = tok_v[pl.ds(q * 16, 16)]
            ddesc.wait()

            def compute_chunk(cc, b, p=p):
                @plsc.parallel_loop(0, NODES_PER_CHUNK, unroll=4)
                def node_body(nl):
                    n = cc * NODES_PER_CHUNK + nl
                    r0 = nl * N_FEAT
                    for q in range(EMB // 16):
                        col = pl.ds(q * 16, 16)
                        acc = rows_v[b, r0, col]
                        for f in range(1, N_FEAT):
                            acc = acc + rows_v[b, r0 + f, col]
                        acc = acc + drows_v[n, col]
                        out_v[p, 1 + n, col] = acc

            idx2 = idxf_v.at[p]

            def gather_chunk(cc, b):
                return pltpu.async_copy(
                    tab_hbm.at[idx2.at[pl.ds(cc * CHUNK, CHUNK)]],
                    rows_v.at[b], gsem[b])

            def chunk_pair(t, _, p=p):
                cc0 = 2 * t
                gather_chunk(cc0 + 1, 1)
                pltpu.make_async_copy(
                    tab_hbm.at[idx2.at[pl.ds(cc0 * CHUNK, CHUNK)]],
                    rows_v.at[0], gsem[0]).wait()
                compute_chunk(cc0, 0)

                @pl.when(t + 1 < N_CHUNK // 2)
                def _():
                    gather_chunk(jnp.minimum(cc0 + 2, N_CHUNK - 1), 0)

                pltpu.make_async_copy(
                    tab_hbm.at[idx2.at[pl.ds((cc0 + 1) * CHUNK, CHUNK)]],
                    rows_v.at[1], gsem[1]).wait()
                compute_chunk(cc0 + 1, 1)
                return _

            lax.fori_loop(0, N_CHUNK // 2, chunk_pair, None)

            odesc[p] = pltpu.async_copy(out_v.at[p], out_hbm.at[g], osem[p])

        odesc[0].wait()
        odesc[1].wait()

    return k(x_idx, deg_idx, node_table, degree_table, graph_token)


def kernel(x, degree, node_table, degree_table, graph_token):
    return _sc_call(x.astype(jnp.int32), degree.astype(jnp.int32),
                    node_table, degree_table, graph_token)


# speedup vs baseline: 1.1336x; 1.1336x over previous
import jax
import jax.numpy as jnp
from jax import lax
from jax.experimental import pallas as pl
from jax.experimental.pallas import tpu as pltpu
from jax.experimental.pallas import tpu_sc as plsc

N_GRAPH, N_NODE, N_FEAT = 256, 256, 8
EMB = 64
NC, NS = 2, 16
NW = NC * NS
GPW = N_GRAPH // NW
ROWS = N_NODE + 1
OUT_ROWS = 264
E = N_NODE * N_FEAT
CHUNK = 256
NODES_PER_CHUNK = CHUNK // N_FEAT
N_CHUNK = E // CHUNK


def _sc_call(x_idx, deg_idx, node_table, degree_table, graph_token):
    mesh = plsc.VectorSubcoreMesh(
        core_axis_name="c", subcore_axis_name="s",
        num_cores=NC, num_subcores=NS)

    @pl.kernel(
        out_type=jax.ShapeDtypeStruct((N_GRAPH, OUT_ROWS, EMB), jnp.float32),
        mesh=mesh,
        scratch_types=[
            pltpu.VMEM((2, N_CHUNK, CHUNK), jnp.int32),
            pltpu.VMEM((2, N_NODE), jnp.int32),
            pltpu.VMEM((2, CHUNK, EMB), jnp.float32),
            pltpu.VMEM((N_NODE, EMB), jnp.float32),
            pltpu.VMEM((EMB,), jnp.float32),
            pltpu.VMEM((2, OUT_ROWS, EMB), jnp.float32),
            pltpu.SemaphoreType.DMA,
            pltpu.SemaphoreType.DMA,
            pltpu.SemaphoreType.DMA,
            pltpu.SemaphoreType.DMA,
            pltpu.SemaphoreType.DMA,
            pltpu.SemaphoreType.DMA,
            pltpu.SemaphoreType.DMA,
        ],
        compiler_params=pltpu.CompilerParams(use_tc_tiling_on_sc=False,
                                             needs_layout_passes=False),
    )
    def k(x_hbm, deg_hbm, tab_hbm, dtab_hbm, tok_hbm, out_hbm,
          idx_v, didx_v, rows_v, drows_v, tok_v, out_v,
          gsem0, gsem1, osem0, osem1, ixsem0, ixsem1, dgsem):
        gsem = (gsem0, gsem1)
        osem = (osem0, osem1)
        ixsem = (ixsem0, ixsem1)

        c = lax.axis_index("c")
        s = lax.axis_index("s")
        wid = c * NS + s

        pltpu.sync_copy(tok_hbm.at[0], tok_v)

        def fetch_idx(i, p):
            g = wid * GPW + i
            a = pltpu.async_copy(x_hbm.at[g], idx_v.at[p], ixsem[p])
            b = pltpu.async_copy(deg_hbm.at[g], didx_v.at[p], ixsem[p])
            return (a, b)

        idesc = [None, None]
        gdesc = [None, None]
        odesc = [None, None]

        idesc[0] = fetch_idx(0, 0)

        for i in range(GPW):
            p = i % 2
            g = wid * GPW + i

            idesc[p][0].wait()
            idesc[p][1].wait()
            ddesc = pltpu.async_copy(dtab_hbm.at[didx_v.at[p]],
                                     drows_v, dgsem)
            gdesc[0] = pltpu.async_copy(
                tab_hbm.at[idx_v.at[p].at[0]], rows_v.at[0], gsem[0])
            if i + 1 < GPW:
                idesc[1 - p] = fetch_idx(i + 1, 1 - p)
            if i >= 2:
                odesc[p].wait()
            for q in range(EMB // 16):
                out_v[p, 0, pl.ds(q * 16, 16)] = tok_v[pl.ds(q * 16, 16)]
            ddesc.wait()

            def compute_chunk(cc, b, p=p):
                @plsc.parallel_loop(0, NODES_PER_CHUNK, unroll=4)
                def node_body(nl):
                    n = cc * NODES_PER_CHUNK + nl
                    r0 = nl * N_FEAT
                    for q in range(EMB // 16):
                        col = pl.ds(q * 16, 16)
                        acc = rows_v[b, r0, col]
                        for f in range(1, N_FEAT):
                            acc = acc + rows_v[b, r0 + f, col]
                        acc = acc + drows_v[n, col]
                        out_v[p, 1 + n, col] = acc

            idx2 = idx_v.at[p]

            def gather_chunk(cc, b):
                return pltpu.async_copy(
                    tab_hbm.at[idx2.at[cc]], rows_v.at[b], gsem[b])

            def chunk_pair(t, _, p=p):
                cc0 = 2 * t
                gather_chunk(cc0 + 1, 1)
                pltpu.make_async_copy(
                    tab_hbm.at[idx2.at[cc0]], rows_v.at[0], gsem[0]).wait()
                compute_chunk(cc0, 0)

                @pl.when(t + 1 < N_CHUNK // 2)
                def _():
                    gather_chunk(jnp.minimum(cc0 + 2, N_CHUNK - 1), 0)

                pltpu.make_async_copy(
                    tab_hbm.at[idx2.at[cc0 + 1]], rows_v.at[1], gsem[1]).wait()
                compute_chunk(cc0 + 1, 1)
                return _

            lax.fori_loop(0, N_CHUNK // 2, chunk_pair, None)

            odesc[p] = pltpu.async_copy(out_v.at[p], out_hbm.at[g], osem[p])

        odesc[0].wait()
        odesc[1].wait()

    return k(x_idx, deg_idx, node_table, degree_table, graph_token)


def kernel(x, degree, node_table, degree_table, graph_token):
    x_idx = x.astype(jnp.int32).reshape(N_GRAPH, N_CHUNK, CHUNK)
    out = _sc_call(x_idx, degree.astype(jnp.int32),
                   node_table, degree_table, graph_token)
    return out[:, :ROWS, :]


# speedup vs baseline: 1.1404x; 1.0060x over previous
import jax
import jax.numpy as jnp
from jax import lax
from jax.experimental import pallas as pl
from jax.experimental.pallas import tpu as pltpu
from jax.experimental.pallas import tpu_sc as plsc

N_GRAPH, N_NODE, N_FEAT = 256, 256, 8
EMB = 64
NC, NS = 2, 16
NW = NC * NS
GPW = N_GRAPH // NW
ROWS = N_NODE + 1
E = N_NODE * N_FEAT
CHUNK = 256
NODES_PER_CHUNK = CHUNK // N_FEAT
N_CHUNK = E // CHUNK


def _sc_call(x_idx, deg_idx, node_table, degree_table, graph_token):
    mesh = plsc.VectorSubcoreMesh(
        core_axis_name="c", subcore_axis_name="s",
        num_cores=NC, num_subcores=NS)

    @pl.kernel(
        out_type=jax.ShapeDtypeStruct((N_GRAPH, ROWS, EMB), jnp.float32),
        mesh=mesh,
        scratch_types=[
            pltpu.VMEM((2, N_CHUNK, CHUNK), jnp.int32),
            pltpu.VMEM((2, N_NODE), jnp.int32),
            pltpu.VMEM((2, CHUNK, EMB), jnp.float32),
            pltpu.VMEM((N_NODE, EMB), jnp.float32),
            pltpu.VMEM((EMB,), jnp.float32),
            pltpu.VMEM((2, ROWS, EMB), jnp.float32),
            pltpu.SemaphoreType.DMA,
            pltpu.SemaphoreType.DMA,
            pltpu.SemaphoreType.DMA,
            pltpu.SemaphoreType.DMA,
            pltpu.SemaphoreType.DMA,
            pltpu.SemaphoreType.DMA,
            pltpu.SemaphoreType.DMA,
        ],
        compiler_params=pltpu.CompilerParams(use_tc_tiling_on_sc=False,
                                             needs_layout_passes=False),
    )
    def k(x_hbm, deg_hbm, tab_hbm, dtab_hbm, tok_hbm, out_hbm,
          idx_v, didx_v, rows_v, drows_v, tok_v, out_v,
          gsem0, gsem1, osem0, osem1, ixsem0, ixsem1, dgsem):
        gsem = (gsem0, gsem1)
        osem = (osem0, osem1)
        ixsem = (ixsem0, ixsem1)

        c = lax.axis_index("c")
        s = lax.axis_index("s")
        wid = c * NS + s

        pltpu.sync_copy(tok_hbm.at[0], tok_v)

        def fetch_idx(i, p):
            g = wid * GPW + i
            a = pltpu.async_copy(x_hbm.at[g], idx_v.at[p], ixsem[p])
            b = pltpu.async_copy(deg_hbm.at[g], didx_v.at[p], ixsem[p])
            return (a, b)

        idesc = [None, None]
        gdesc = [None, None]
        odesc = [None, None]

        idesc[0] = fetch_idx(0, 0)

        for i in range(GPW):
            p = i % 2
            g = wid * GPW + i

            idesc[p][0].wait()
            idesc[p][1].wait()
            ddesc = pltpu.async_copy(dtab_hbm.at[didx_v.at[p]],
                                     drows_v, dgsem)
            gdesc[0] = pltpu.async_copy(
                tab_hbm.at[idx_v.at[p].at[0]], rows_v.at[0], gsem[0])
            if i + 1 < GPW:
                idesc[1 - p] = fetch_idx(i + 1, 1 - p)
            if i >= 2:
                odesc[p].wait()
            for q in range(EMB // 16):
                out_v[p, 0, pl.ds(q * 16, 16)] = tok_v[pl.ds(q * 16, 16)]
            ddesc.wait()

            def compute_chunk(cc, b, p=p):
                @plsc.parallel_loop(0, NODES_PER_CHUNK, unroll=8)
                def node_body(nl):
                    n = cc * NODES_PER_CHUNK + nl
                    r0 = nl * N_FEAT
                    for q in range(EMB // 16):
                        col = pl.ds(q * 16, 16)
                        acc = rows_v[b, r0, col]
                        for f in range(1, N_FEAT):
                            acc = acc + rows_v[b, r0 + f, col]
                        acc = acc + drows_v[n, col]
                        out_v[p, 1 + n, col] = acc

            idx2 = idx_v.at[p]

            def gather_chunk(cc, b):
                return pltpu.async_copy(
                    tab_hbm.at[idx2.at[cc]], rows_v.at[b], gsem[b])

            def chunk_pair(t, _, p=p):
                cc0 = 2 * t
                gather_chunk(cc0 + 1, 1)
                pltpu.make_async_copy(
                    tab_hbm.at[idx2.at[cc0]], rows_v.at[0], gsem[0]).wait()
                compute_chunk(cc0, 0)

                @pl.when(t + 1 < N_CHUNK // 2)
                def _():
                    gather_chunk(jnp.minimum(cc0 + 2, N_CHUNK - 1), 0)

                pltpu.make_async_copy(
                    tab_hbm.at[idx2.at[cc0 + 1]], rows_v.at[1], gsem[1]).wait()
                compute_chunk(cc0 + 1, 1)
                return _

            lax.fori_loop(0, N_CHUNK // 2, chunk_pair, None)

            odesc[p] = pltpu.async_copy(out_v.at[p], out_hbm.at[g], osem[p])

        odesc[0].wait()
        odesc[1].wait()

    return k(x_idx, deg_idx, node_table, degree_table, graph_token)


def kernel(x, degree, node_table, degree_table, graph_token):
    x_idx = x.astype(jnp.int32).reshape(N_GRAPH, N_CHUNK, CHUNK)
    return _sc_call(x_idx, degree.astype(jnp.int32),
                    node_table, degree_table, graph_token)


# speedup vs baseline: 1.1473x; 1.0061x over previous
import jax
import jax.numpy as jnp
from jax import lax
from jax.experimental import pallas as pl
from jax.experimental.pallas import tpu as pltpu
from jax.experimental.pallas import tpu_sc as plsc

N_GRAPH, N_NODE, N_FEAT = 256, 256, 8
EMB = 64
NC, NS = 2, 16
NW = NC * NS
GPW = N_GRAPH // NW
ROWS = N_NODE + 1
E = N_NODE * N_FEAT
CHUNK = 256
NODES_PER_CHUNK = CHUNK // N_FEAT
N_CHUNK = E // CHUNK


def _sc_call(x_idx, deg_idx, node_table, degree_table, graph_token):
    mesh = plsc.VectorSubcoreMesh(
        core_axis_name="c", subcore_axis_name="s",
        num_cores=NC, num_subcores=NS)

    @pl.kernel(
        out_type=jax.ShapeDtypeStruct((N_GRAPH, ROWS, EMB), jnp.float32),
        mesh=mesh,
        scratch_types=[
            pltpu.VMEM((2, N_CHUNK, CHUNK), jnp.int32),
            pltpu.VMEM((2, N_NODE), jnp.int32),
            pltpu.VMEM((2, CHUNK, EMB), jnp.float32),
            pltpu.VMEM((N_NODE, EMB), jnp.float32),
            pltpu.VMEM((EMB,), jnp.float32),
            pltpu.VMEM((2, ROWS, EMB), jnp.float32),
            pltpu.SemaphoreType.DMA,
            pltpu.SemaphoreType.DMA,
            pltpu.SemaphoreType.DMA,
            pltpu.SemaphoreType.DMA,
            pltpu.SemaphoreType.DMA,
            pltpu.SemaphoreType.DMA,
            pltpu.SemaphoreType.DMA,
        ],
        compiler_params=pltpu.CompilerParams(use_tc_tiling_on_sc=False,
                                             needs_layout_passes=False),
    )
    def k(x_hbm, deg_hbm, tab_hbm, dtab_hbm, tok_hbm, out_hbm,
          idx_v, didx_v, rows_v, drows_v, tok_v, out_v,
          gsem0, gsem1, osem0, osem1, ixsem0, ixsem1, dgsem):
        gsem = (gsem0, gsem1)
        osem = (osem0, osem1)
        ixsem = (ixsem0, ixsem1)

        c = lax.axis_index("c")
        s = lax.axis_index("s")
        wid = c * NS + s

        pltpu.sync_copy(tok_hbm.at[0], tok_v)

        def fetch_idx(i, p):
            g = wid * GPW + i
            a = pltpu.async_copy(x_hbm.at[g], idx_v.at[p], ixsem[p])
            b = pltpu.async_copy(deg_hbm.at[g], didx_v.at[p], ixsem[p])
            return (a, b)

        idesc = [None, None]
        gdesc = [None, None]
        odesc = [None, None]

        idesc[0] = fetch_idx(0, 0)

        for i in range(GPW):
            p = i % 2
            g = wid * GPW + i

            idesc[p][0].wait()
            idesc[p][1].wait()
            ddesc = pltpu.async_copy(dtab_hbm.at[didx_v.at[p]],
                                     drows_v, dgsem)
            gdesc[0] = pltpu.async_copy(
                tab_hbm.at[idx_v.at[p].at[0]], rows_v.at[0], gsem[0])
            if i + 1 < GPW:
                idesc[1 - p] = fetch_idx(i + 1, 1 - p)
            if i >= 2:
                odesc[p].wait()
            for q in range(EMB // 16):
                out_v[p, 0, pl.ds(q * 16, 16)] = tok_v[pl.ds(q * 16, 16)]
            ddesc.wait()

            def compute_chunk(cc, b, p=p):
                @plsc.parallel_loop(0, NODES_PER_CHUNK, unroll=4)
                def node_body(nl):
                    n = cc * NODES_PER_CHUNK + nl
                    r0 = nl * N_FEAT
                    for q in range(EMB // 16):
                        col = pl.ds(q * 16, 16)
                        acc = rows_v[b, r0, col]
                        for f in range(1, N_FEAT):
                            acc = acc + rows_v[b, r0 + f, col]
                        acc = acc + drows_v[n, col]
                        out_v[p, 1 + n, col] = acc

            idx2 = idx_v.at[p]

            def gather_chunk(cc, b):
                return pltpu.async_copy(
                    tab_hbm.at[idx2.at[cc]], rows_v.at[b], gsem[b])

            def chunk_pair(t, _, p=p):
                cc0 = 2 * t
                gather_chunk(cc0 + 1, 1)
                pltpu.make_async_copy(
                    tab_hbm.at[idx2.at[cc0]], rows_v.at[0], gsem[0]).wait()
                compute_chunk(cc0, 0)

                @pl.when(t + 1 < N_CHUNK // 2)
                def _():
                    gather_chunk(jnp.minimum(cc0 + 2, N_CHUNK - 1), 0)

                pltpu.make_async_copy(
                    tab_hbm.at[idx2.at[cc0 + 1]], rows_v.at[1], gsem[1]).wait()
                compute_chunk(cc0 + 1, 1)
                return _

            lax.fori_loop(0, N_CHUNK // 2, chunk_pair, None)

            odesc[p] = pltpu.async_copy(out_v.at[p], out_hbm.at[g], osem[p])

        odesc[0].wait()
        odesc[1].wait()

    return k(x_idx, deg_idx, node_table, degree_table, graph_token)


def kernel(x, degree, node_table, degree_table, graph_token):
    x_idx = x.astype(jnp.int32).reshape(N_GRAPH, N_CHUNK, CHUNK)
    return _sc_call(x_idx, degree.astype(jnp.int32),
                    node_table, degree_table, graph_token)
